# baseline (speedup 1.0000x reference)
import functools

import jax
import jax.numpy as jnp
from jax import lax
from jax.experimental import pallas as pl
from jax.experimental.pallas import tpu as pltpu
from jax.experimental.pallas import tpu_sc as plsc

N = 10000
E = 320000
IN_DIM = 128
D = 64
T = 4
L = 2
C = 16
VTH = 1.0
STEP = 0.1

CH = 128
NTILES = 16
RPT = 640
RCH = 80
NPAD = RPT * NTILES
BLK = 8
ECH_ENC = 1280
ECH_LAY = 2560
FAKE_ENC = ECH_ENC * CH - E // 2
FAKE_LAY = ECH_LAY * CH - E


def _spike(v):
    s = (v >= VTH).astype(v.dtype)
    sg = jax.nn.sigmoid(4.0 * (v - VTH))
    return sg + (s - sg)


BN = 1000
GRID = N // BN


def _enc_matmul_body(f_ref, w_ref, b_ref, o_ref):
    h = jnp.dot(f_ref[...], w_ref[...], preferred_element_type=jnp.float32) + b_ref[...]
    pad = jnp.concatenate(
        [jnp.ones((h.shape[0], 1), jnp.float32),
         jnp.zeros((h.shape[0], D - 1), jnp.float32)], axis=1)
    o_ref[...] = jnp.concatenate([h, pad], axis=1)


def _enc_matmul(features, W_enc, b_enc):
    return pl.pallas_call(
        _enc_matmul_body,
        grid=(GRID,),
        in_specs=[
            pl.BlockSpec((BN, IN_DIM), lambda i: (i, 0)),
            pl.BlockSpec((IN_DIM, D), lambda i: (0, 0)),
            pl.BlockSpec((1, D), lambda i: (0, 0)),
        ],
        out_specs=pl.BlockSpec((BN, 2 * D), lambda i: (i, 0)),
        out_shape=jax.ShapeDtypeStruct((N, 2 * D), jnp.float32),
    )(features, W_enc, b_enc.reshape(1, D))


def _enc_if_body(sums_ref, x2_ref, z_ref, dinv_ref):
    deg = sums_ref[0][:, D:D + 1] + sums_ref[1][:, D:D + 1]
    dinv = 1.0 / jnp.maximum(deg, 1.0)
    agg = (sums_ref[0][:, :D] + sums_ref[1][:, :D]) * dinv
    v = jnp.zeros_like(agg)
    sp = []
    for _ in range(T):
        v = v + agg
        s = _spike(v)
        v = v - s * VTH
        sp.append(s)
    x2_ref[0] = jnp.concatenate([sp[0], sp[1]], axis=1)
    x2_ref[1] = jnp.concatenate([sp[2], sp[3]], axis=1)
    z_ref[...] = STEP * (sp[0] + sp[1] + sp[2] + sp[3])
    dinv_ref[...] = dinv


def _enc_if(sums):
    return pl.pallas_call(
        _enc_if_body,
        grid=(GRID,),
        in_specs=[
            pl.BlockSpec((2, BN, IN_DIM), lambda i: (0, i, 0)),
        ],
        out_specs=[
            pl.BlockSpec((2, BN, 2 * D), lambda i: (0, i, 0)),
            pl.BlockSpec((BN, D), lambda i: (i, 0)),
            pl.BlockSpec((BN, 1), lambda i: (i, 0)),
        ],
        out_shape=[
            jax.ShapeDtypeStruct((2, N, 2 * D), jnp.float32),
            jax.ShapeDtypeStruct((N, D), jnp.float32),
            jax.ShapeDtypeStruct((N, 1), jnp.float32),
        ],
    )(sums)


def _layer_body(m2_ref, dinv_ref, w_ref, b_ref, z_ref, x2_ref, zn_ref):
    dinv = dinv_ref[...]
    v = jnp.zeros((BN, D), jnp.float32)
    sp = []
    for t in range(T):
        m_t = m2_ref[t // 2][:, (t % 2) * D:(t % 2 + 1) * D] * dinv
        u = jnp.dot(m_t, w_ref[...], preferred_element_type=jnp.float32) + b_ref[...]
        v = v + u
        s = _spike(v)
        v = v - s * VTH
        sp.append(s)
    x2_ref[0] = jnp.concatenate([sp[0], sp[1]], axis=1)
    x2_ref[1] = jnp.concatenate([sp[2], sp[3]], axis=1)
    zn_ref[...] = z_ref[...] + STEP * (sp[0] + sp[1] + sp[2] + sp[3])


def _layer_tc(m2, dinv, W, b, z):
    return pl.pallas_call(
        _layer_body,
        grid=(GRID,),
        in_specs=[
            pl.BlockSpec((2, BN, 2 * D), lambda i: (0, i, 0)),
            pl.BlockSpec((BN, 1), lambda i: (i, 0)),
            pl.BlockSpec((D, D), lambda i: (0, 0)),
            pl.BlockSpec((1, D), lambda i: (0, 0)),
            pl.BlockSpec((BN, D), lambda i: (i, 0)),
        ],
        out_specs=[
            pl.BlockSpec((2, BN, 2 * D), lambda i: (0, i, 0)),
            pl.BlockSpec((BN, D), lambda i: (i, 0)),
        ],
        out_shape=[
            jax.ShapeDtypeStruct((2, N, 2 * D), jnp.float32),
            jax.ShapeDtypeStruct((N, D), jnp.float32),
        ],
    )(m2, dinv, W, b.reshape(1, D), z)


def _final_body(m2_ref, dinv_ref, w_ref, b_ref, z_ref, fcw_ref, fcb_ref, o_ref):
    dinv = dinv_ref[...]
    v = jnp.zeros((BN, D), jnp.float32)
    acc = jnp.zeros((BN, D), jnp.float32)
    for t in range(T):
        m_t = m2_ref[t // 2][:, (t % 2) * D:(t % 2 + 1) * D] * dinv
        u = jnp.dot(m_t, w_ref[...], preferred_element_type=jnp.float32) + b_ref[...]
        v = v + u
        s = _spike(v)
        v = v - s * VTH
        acc = acc + s
    zf = z_ref[...] + STEP * acc
    o_ref[...] = (
        jnp.dot(zf, fcw_ref[...], preferred_element_type=jnp.float32) + fcb_ref[...]
    )


def _final_tc(m2, dinv, W, b, z, fc_W, fc_b):
    return pl.pallas_call(
        _final_body,
        grid=(GRID,),
        in_specs=[
            pl.BlockSpec((2, BN, 2 * D), lambda i: (0, i, 0)),
            pl.BlockSpec((BN, 1), lambda i: (i, 0)),
            pl.BlockSpec((D, D), lambda i: (0, 0)),
            pl.BlockSpec((1, D), lambda i: (0, 0)),
            pl.BlockSpec((BN, D), lambda i: (i, 0)),
            pl.BlockSpec((D, C), lambda i: (0, 0)),
            pl.BlockSpec((1, C), lambda i: (0, 0)),
        ],
        out_specs=pl.BlockSpec((BN, C), lambda i: (i, 0)),
        out_shape=jax.ShapeDtypeStruct((N, C), jnp.float32),
    )(m2, dinv, W, b.reshape(1, D), z, fc_W, fc_b.reshape(1, C))


_MESH = plsc.VectorSubcoreMesh(core_axis_name="c", subcore_axis_name="s")


def _zero_stripe(shared, s, tbuf):
    def body(k, _):
        pltpu.sync_copy(tbuf, shared.at[pl.ds(s * RPT + k * RCH, RCH)])
        return 0

    lax.fori_loop(0, jnp.where(s == NTILES - 1, 5, 8), body, 0)


def _stripe_writeback(shared, hbm, s, tbuf):
    def body(k, _):
        off = s * RPT + k * RCH
        pltpu.sync_copy(shared.at[pl.ds(off, RCH)], tbuf)
        pltpu.sync_copy(tbuf, hbm.at[pl.ds(off, RCH)])
        return 0

    lax.fori_loop(0, jnp.where(s == NTILES - 1, 5, 8), body, 0)


def _pipelined_block(table, accum, sblk, dblk, rows, sems):
    gsem, ssem = sems
    gd = {0: pltpu.async_copy(table.at[sblk.at[0]], rows[0], gsem)}
    sd = {}
    for k in range(BLK):
        b = k & 1
        gd[k].wait()
        sd[k] = pltpu.async_copy(rows[b], accum.at[dblk.at[k]], ssem, add=True)
        if k + 1 < BLK:
            if k >= 1:
                sd[k - 1].wait()
            gd[k + 1] = pltpu.async_copy(table.at[sblk.at[k + 1]], rows[1 - b],
                                         gsem)
    sd[BLK - 2].wait()
    sd[BLK - 1].wait()


@functools.partial(
    pl.kernel,
    mesh=_MESH,
    out_type=jax.ShapeDtypeStruct((2, N, IN_DIM), jnp.float32),
    scratch_types=[
        pltpu.VMEM((BLK, CH), jnp.int32),
        pltpu.VMEM((BLK, CH), jnp.int32),
        pltpu.VMEM((CH, IN_DIM), jnp.float32),
        pltpu.VMEM((CH, IN_DIM), jnp.float32),
        pltpu.VMEM((RCH, IN_DIM), jnp.float32),
        pltpu.VMEM_SHARED((N + BLK, IN_DIM), jnp.float32),
        pltpu.SemaphoreType.DMA,
        pltpu.SemaphoreType.DMA,
    ],
)
def _enc_agg_sc(f_hbm, src_hbm, dst_hbm, zrow_hbm, sums_hbm,
                sblk, dblk, rows0, rows1, tbuf, accum, sem0, sem1):
    c = lax.axis_index("c")
    s = lax.axis_index("s")

    pltpu.sync_copy(zrow_hbm, tbuf)
    _zero_stripe(accum, s, tbuf)
    plsc.subcore_barrier()

    def body(blk, _):
        base = c * ECH_ENC + s * 80 + blk * BLK
        pltpu.sync_copy(src_hbm.at[pl.ds(base, BLK)], sblk)
        pltpu.sync_copy(dst_hbm.at[pl.ds(base, BLK)], dblk)
        _pipelined_block(f_hbm, accum, sblk, dblk, (rows0, rows1),
                         (sem0, sem1))
        return 0

    lax.fori_loop(0, 10, body, 0)
    plsc.subcore_barrier()

    _stripe_writeback(accum, sums_hbm.at[c], s, tbuf)


@functools.partial(
    pl.kernel,
    mesh=_MESH,
    out_type=jax.ShapeDtypeStruct((2, N, 2 * D), jnp.float32),
    scratch_types=[
        pltpu.VMEM((BLK, CH), jnp.int32),
        pltpu.VMEM((BLK, CH), jnp.int32),
        pltpu.VMEM((CH, 2 * D), jnp.float32),
        pltpu.VMEM((CH, 2 * D), jnp.float32),
        pltpu.VMEM((RCH, 2 * D), jnp.float32),
        pltpu.VMEM_SHARED((N + BLK, 2 * D), jnp.float32),
        pltpu.SemaphoreType.DMA,
        pltpu.SemaphoreType.DMA,
    ],
)
def _layer_agg_sc(x2_hbm, src_hbm, dst_hbm, zrow_hbm, m2_hbm,
                  sblk, dblk, rows0, rows1, tbuf, accum, sem0, sem1):
    c = lax.axis_index("c")
    s = lax.axis_index("s")

    pltpu.sync_copy(zrow_hbm, tbuf)
    _zero_stripe(accum, s, tbuf)
    plsc.subcore_barrier()

    def body(blk, _):
        base = s * 160 + blk * BLK
        pltpu.sync_copy(src_hbm.at[pl.ds(base, BLK)], sblk)
        pltpu.sync_copy(dst_hbm.at[pl.ds(base, BLK)], dblk)
        _pipelined_block(x2_hbm.at[c], accum, sblk, dblk, (rows0, rows1),
                         (sem0, sem1))
        return 0

    lax.fori_loop(0, 20, body, 0)
    plsc.subcore_barrier()

    _stripe_writeback(accum, m2_hbm.at[c], s, tbuf)


@jax.jit
def kernel(features, edge_index, W_enc, b_enc, W_layers, b_layers, fc_W, fc_b):
    src = edge_index[0]
    dst = edge_index[1]
    zrow = jnp.zeros((RCH, IN_DIM), jnp.float32)

    fsrc_e = jnp.arange(FAKE_ENC, dtype=jnp.int32) % N
    fdst_e = N + jnp.arange(FAKE_ENC, dtype=jnp.int32) % BLK
    e2 = E // 2
    esrc = jnp.concatenate([src[:e2], fsrc_e, src[e2:], fsrc_e]).reshape(-1, CH)
    edst = jnp.concatenate([dst[:e2], fdst_e, dst[e2:], fdst_e]).reshape(-1, CH)
    fsrc_l = jnp.arange(FAKE_LAY, dtype=jnp.int32) % N
    fdst_l = N + jnp.arange(FAKE_LAY, dtype=jnp.int32) % BLK
    lsrc = jnp.concatenate([src, fsrc_l]).reshape(-1, CH)
    ldst = jnp.concatenate([dst, fdst_l]).reshape(-1, CH)

    h = _enc_matmul(features, W_enc, b_enc)
    sums = _enc_agg_sc(h, esrc, edst, zrow)
    x2, z, dinv = _enc_if(sums)

    m2 = _layer_agg_sc(x2, lsrc, ldst, zrow)
    x2, z = _layer_tc(m2, dinv, W_layers[0], b_layers[0], z)

    m2 = _layer_agg_sc(x2, lsrc, ldst, zrow)
    out = _final_tc(m2, dinv, W_layers[1], b_layers[1], z, fc_W, fc_b)
    return out


# speedup vs baseline: 1.0545x; 1.0545x over previous
import functools

import jax
import jax.numpy as jnp
from jax import lax
from jax.experimental import pallas as pl
from jax.experimental.pallas import tpu as pltpu
from jax.experimental.pallas import tpu_sc as plsc

N = 10000
E = 320000
IN_DIM = 128
D = 64
T = 4
L = 2
C = 16
VTH = 1.0
STEP = 0.1

CH = 128
NTILES = 16
RPT = 640
RCH = 80
NPAD = RPT * NTILES
BLK = 8
ECH_ENC = 1280
ECH_LAY = 2560
FAKE_ENC = ECH_ENC * CH - E // 2
FAKE_LAY = ECH_LAY * CH - E


def _spike(v):
    s = (v >= VTH).astype(v.dtype)
    sg = jax.nn.sigmoid(4.0 * (v - VTH))
    return sg + (s - sg)


BN = 1000
GRID = N // BN


def _enc_matmul_body(f_ref, w_ref, b_ref, o_ref):
    h = jnp.dot(f_ref[...], w_ref[...], preferred_element_type=jnp.float32) + b_ref[...]
    pad = jnp.concatenate(
        [jnp.ones((h.shape[0], 1), jnp.float32),
         jnp.zeros((h.shape[0], D - 1), jnp.float32)], axis=1)
    o_ref[...] = jnp.concatenate([h, pad], axis=1)


def _enc_matmul(features, W_enc, b_enc):
    return pl.pallas_call(
        _enc_matmul_body,
        grid=(GRID,),
        in_specs=[
            pl.BlockSpec((BN, IN_DIM), lambda i: (i, 0)),
            pl.BlockSpec((IN_DIM, D), lambda i: (0, 0)),
            pl.BlockSpec((1, D), lambda i: (0, 0)),
        ],
        out_specs=pl.BlockSpec((BN, 2 * D), lambda i: (i, 0)),
        out_shape=jax.ShapeDtypeStruct((N, 2 * D), jnp.float32),
    )(features, W_enc, b_enc.reshape(1, D))


def _enc_if_body(sums_ref, x2_ref, z_ref, dinv_ref):
    deg = sums_ref[0][:, D:D + 1] + sums_ref[1][:, D:D + 1]
    dinv = 1.0 / jnp.maximum(deg, 1.0)
    agg = (sums_ref[0][:, :D] + sums_ref[1][:, :D]) * dinv
    v = jnp.zeros_like(agg)
    sp = []
    for _ in range(T):
        v = v + agg
        s = _spike(v)
        v = v - s * VTH
        sp.append(s)
    x2_ref[0] = jnp.concatenate([sp[0], sp[1]], axis=1)
    x2_ref[1] = jnp.concatenate([sp[2], sp[3]], axis=1)
    z_ref[...] = STEP * (sp[0] + sp[1] + sp[2] + sp[3])
    dinv_ref[...] = dinv


def _enc_if(sums):
    return pl.pallas_call(
        _enc_if_body,
        grid=(GRID,),
        in_specs=[
            pl.BlockSpec((2, BN, IN_DIM), lambda i: (0, i, 0)),
        ],
        out_specs=[
            pl.BlockSpec((2, BN, 2 * D), lambda i: (0, i, 0)),
            pl.BlockSpec((BN, D), lambda i: (i, 0)),
            pl.BlockSpec((BN, 1), lambda i: (i, 0)),
        ],
        out_shape=[
            jax.ShapeDtypeStruct((2, N, 2 * D), jnp.float32),
            jax.ShapeDtypeStruct((N, D), jnp.float32),
            jax.ShapeDtypeStruct((N, 1), jnp.float32),
        ],
    )(sums)


def _layer_body(m2_ref, dinv_ref, w_ref, b_ref, z_ref, x2_ref, zn_ref):
    dinv = dinv_ref[...]
    v = jnp.zeros((BN, D), jnp.float32)
    sp = []
    for t in range(T):
        m_t = m2_ref[t // 2][:, (t % 2) * D:(t % 2 + 1) * D] * dinv
        u = jnp.dot(m_t, w_ref[...], preferred_element_type=jnp.float32) + b_ref[...]
        v = v + u
        s = _spike(v)
        v = v - s * VTH
        sp.append(s)
    x2_ref[0] = jnp.concatenate([sp[0], sp[1]], axis=1)
    x2_ref[1] = jnp.concatenate([sp[2], sp[3]], axis=1)
    zn_ref[...] = z_ref[...] + STEP * (sp[0] + sp[1] + sp[2] + sp[3])


def _layer_tc(m2, dinv, W, b, z):
    return pl.pallas_call(
        _layer_body,
        grid=(GRID,),
        in_specs=[
            pl.BlockSpec((2, BN, 2 * D), lambda i: (0, i, 0)),
            pl.BlockSpec((BN, 1), lambda i: (i, 0)),
            pl.BlockSpec((D, D), lambda i: (0, 0)),
            pl.BlockSpec((1, D), lambda i: (0, 0)),
            pl.BlockSpec((BN, D), lambda i: (i, 0)),
        ],
        out_specs=[
            pl.BlockSpec((2, BN, 2 * D), lambda i: (0, i, 0)),
            pl.BlockSpec((BN, D), lambda i: (i, 0)),
        ],
        out_shape=[
            jax.ShapeDtypeStruct((2, N, 2 * D), jnp.float32),
            jax.ShapeDtypeStruct((N, D), jnp.float32),
        ],
    )(m2, dinv, W, b.reshape(1, D), z)


def _final_body(m2_ref, dinv_ref, w_ref, b_ref, z_ref, fcw_ref, fcb_ref, o_ref):
    dinv = dinv_ref[...]
    v = jnp.zeros((BN, D), jnp.float32)
    acc = jnp.zeros((BN, D), jnp.float32)
    for t in range(T):
        m_t = m2_ref[t // 2][:, (t % 2) * D:(t % 2 + 1) * D] * dinv
        u = jnp.dot(m_t, w_ref[...], preferred_element_type=jnp.float32) + b_ref[...]
        v = v + u
        s = _spike(v)
        v = v - s * VTH
        acc = acc + s
    zf = z_ref[...] + STEP * acc
    o_ref[...] = (
        jnp.dot(zf, fcw_ref[...], preferred_element_type=jnp.float32) + fcb_ref[...]
    )


def _final_tc(m2, dinv, W, b, z, fc_W, fc_b):
    return pl.pallas_call(
        _final_body,
        grid=(GRID,),
        in_specs=[
            pl.BlockSpec((2, BN, 2 * D), lambda i: (0, i, 0)),
            pl.BlockSpec((BN, 1), lambda i: (i, 0)),
            pl.BlockSpec((D, D), lambda i: (0, 0)),
            pl.BlockSpec((1, D), lambda i: (0, 0)),
            pl.BlockSpec((BN, D), lambda i: (i, 0)),
            pl.BlockSpec((D, C), lambda i: (0, 0)),
            pl.BlockSpec((1, C), lambda i: (0, 0)),
        ],
        out_specs=pl.BlockSpec((BN, C), lambda i: (i, 0)),
        out_shape=jax.ShapeDtypeStruct((N, C), jnp.float32),
    )(m2, dinv, W, b.reshape(1, D), z, fc_W, fc_b.reshape(1, C))


_MESH = plsc.VectorSubcoreMesh(core_axis_name="c", subcore_axis_name="s")


def _zero_stripe(shared, s, tbuf):
    def body(k, _):
        pltpu.sync_copy(tbuf, shared.at[pl.ds(s * RPT + k * RCH, RCH)])
        return 0

    lax.fori_loop(0, jnp.where(s == NTILES - 1, 5, 8), body, 0)


def _stripe_writeback(shared, hbm, s, tbuf):
    def body(k, _):
        off = s * RPT + k * RCH
        pltpu.sync_copy(shared.at[pl.ds(off, RCH)], tbuf)
        pltpu.sync_copy(tbuf, hbm.at[pl.ds(off, RCH)])
        return 0

    lax.fori_loop(0, jnp.where(s == NTILES - 1, 5, 8), body, 0)


def _block_pair(table, accum, src_hbm, dst_hbm, base, idx, rows, sems, isem):
    sblk0, dblk0, sblk1, dblk1 = idx
    pltpu.sync_copy(src_hbm.at[pl.ds(base, BLK)], sblk0)
    pltpu.sync_copy(dst_hbm.at[pl.ds(base, BLK)], dblk0)
    pf = [pltpu.async_copy(src_hbm.at[pl.ds(base + BLK, BLK)], sblk1, isem),
          pltpu.async_copy(dst_hbm.at[pl.ds(base + BLK, BLK)], dblk1, isem)]
    _pipelined_block(table, accum, sblk0, dblk0, rows, sems)
    pf[0].wait()
    pf[1].wait()
    _pipelined_block(table, accum, sblk1, dblk1, rows, sems)


def _pipelined_block(table, accum, sblk, dblk, rows, sems):
    gsem, _ = sems
    for half in range(BLK // 2):
        gd = [pltpu.async_copy(table.at[sblk.at[2 * half + k]], rows[k], gsem)
              for k in range(2)]
        for k in range(2):
            gd[k].wait()
            pltpu.sync_copy(rows[k], accum.at[dblk.at[2 * half + k]], add=True)


@functools.partial(
    pl.kernel,
    mesh=_MESH,
    out_type=jax.ShapeDtypeStruct((2, N, IN_DIM), jnp.float32),
    scratch_types=[
        pltpu.VMEM((BLK, CH), jnp.int32),
        pltpu.VMEM((BLK, CH), jnp.int32),
        pltpu.VMEM((BLK, CH), jnp.int32),
        pltpu.VMEM((BLK, CH), jnp.int32),
        pltpu.VMEM((CH, IN_DIM), jnp.float32),
        pltpu.VMEM((CH, IN_DIM), jnp.float32),
        pltpu.VMEM((RCH, IN_DIM), jnp.float32),
        pltpu.VMEM_SHARED((N + BLK, IN_DIM), jnp.float32),
        pltpu.SemaphoreType.DMA,
        pltpu.SemaphoreType.DMA,
        pltpu.SemaphoreType.DMA,
    ],
)
def _enc_agg_sc(f_hbm, src_hbm, dst_hbm, zrow_hbm, sums_hbm,
                sblk0, dblk0, sblk1, dblk1, rows0, rows1, tbuf, accum,
                sem0, sem1, isem):
    c = lax.axis_index("c")
    s = lax.axis_index("s")

    pltpu.sync_copy(zrow_hbm, tbuf)
    _zero_stripe(accum, s, tbuf)
    plsc.subcore_barrier()

    def body(pair, _):
        base = c * ECH_ENC + s * 80 + pair * 2 * BLK
        _block_pair(f_hbm, accum, src_hbm, dst_hbm, base,
                    (sblk0, dblk0, sblk1, dblk1), (rows0, rows1),
                    (sem0, sem1), isem)
        return 0

    lax.fori_loop(0, 5, body, 0)
    plsc.subcore_barrier()

    _stripe_writeback(accum, sums_hbm.at[c], s, tbuf)


@functools.partial(
    pl.kernel,
    mesh=_MESH,
    out_type=jax.ShapeDtypeStruct((2, N, 2 * D), jnp.float32),
    scratch_types=[
        pltpu.VMEM((BLK, CH), jnp.int32),
        pltpu.VMEM((BLK, CH), jnp.int32),
        pltpu.VMEM((BLK, CH), jnp.int32),
        pltpu.VMEM((BLK, CH), jnp.int32),
        pltpu.VMEM((CH, 2 * D), jnp.float32),
        pltpu.VMEM((CH, 2 * D), jnp.float32),
        pltpu.VMEM((RCH, 2 * D), jnp.float32),
        pltpu.VMEM_SHARED((N + BLK, 2 * D), jnp.float32),
        pltpu.SemaphoreType.DMA,
        pltpu.SemaphoreType.DMA,
        pltpu.SemaphoreType.DMA,
    ],
)
def _layer_agg_sc(x2_hbm, src_hbm, dst_hbm, zrow_hbm, m2_hbm,
                  sblk0, dblk0, sblk1, dblk1, rows0, rows1, tbuf, accum,
                  sem0, sem1, isem):
    c = lax.axis_index("c")
    s = lax.axis_index("s")

    pltpu.sync_copy(zrow_hbm, tbuf)
    _zero_stripe(accum, s, tbuf)
    plsc.subcore_barrier()

    def body(pair, _):
        base = s * 160 + pair * 2 * BLK
        _block_pair(x2_hbm.at[c], accum, src_hbm, dst_hbm, base,
                    (sblk0, dblk0, sblk1, dblk1), (rows0, rows1),
                    (sem0, sem1), isem)
        return 0

    lax.fori_loop(0, 10, body, 0)
    plsc.subcore_barrier()

    _stripe_writeback(accum, m2_hbm.at[c], s, tbuf)


@jax.jit
def kernel(features, edge_index, W_enc, b_enc, W_layers, b_layers, fc_W, fc_b):
    src = edge_index[0]
    dst = edge_index[1]
    zrow = jnp.zeros((RCH, IN_DIM), jnp.float32)

    fsrc_e = jnp.arange(FAKE_ENC, dtype=jnp.int32) % N
    fdst_e = N + jnp.arange(FAKE_ENC, dtype=jnp.int32) % BLK
    e2 = E // 2
    esrc = jnp.concatenate([src[:e2], fsrc_e, src[e2:], fsrc_e]).reshape(-1, CH)
    edst = jnp.concatenate([dst[:e2], fdst_e, dst[e2:], fdst_e]).reshape(-1, CH)
    fsrc_l = jnp.arange(FAKE_LAY, dtype=jnp.int32) % N
    fdst_l = N + jnp.arange(FAKE_LAY, dtype=jnp.int32) % BLK
    lsrc = jnp.concatenate([src, fsrc_l]).reshape(-1, CH)
    ldst = jnp.concatenate([dst, fdst_l]).reshape(-1, CH)

    h = _enc_matmul(features, W_enc, b_enc)
    sums = _enc_agg_sc(h, esrc, edst, zrow)
    x2, z, dinv = _enc_if(sums)

    m2 = _layer_agg_sc(x2, lsrc, ldst, zrow)
    x2, z = _layer_tc(m2, dinv, W_layers[0], b_layers[0], z)

    m2 = _layer_agg_sc(x2, lsrc, ldst, zrow)
    out = _final_tc(m2, dinv, W_layers[1], b_layers[1], z, fc_W, fc_b)
    return out


# speedup vs baseline: 1.0672x; 1.0120x over previous
import functools

import jax
import jax.numpy as jnp
from jax import lax
from jax.experimental import pallas as pl
from jax.experimental.pallas import tpu as pltpu
from jax.experimental.pallas import tpu_sc as plsc

N = 10000
E = 320000
IN_DIM = 128
D = 64
T = 4
L = 2
C = 16
VTH = 1.0
STEP = 0.1

CH = 128
NTILES = 16
RPT = 640
RCH = 80
NPAD = RPT * NTILES
BLK = 8
ECH_ENC = 1280
ECH_LAY = 2560
FAKE_LAY = ECH_LAY * CH - E


def _spike(v):
    s = (v >= VTH).astype(v.dtype)
    sg = jax.nn.sigmoid(4.0 * (v - VTH))
    return sg + (s - sg)


BN = 1000
GRID = N // BN


def _enc_matmul_body(f_ref, w_ref, b_ref, o_ref):
    h = jnp.dot(f_ref[...], w_ref[...], preferred_element_type=jnp.float32) + b_ref[...]
    pad = jnp.concatenate(
        [jnp.ones((h.shape[0], 1), jnp.float32),
         jnp.zeros((h.shape[0], D - 1), jnp.float32)], axis=1)
    o_ref[...] = jnp.concatenate([h, pad], axis=1)


def _enc_matmul(features, W_enc, b_enc):
    return pl.pallas_call(
        _enc_matmul_body,
        grid=(GRID,),
        in_specs=[
            pl.BlockSpec((BN, IN_DIM), lambda i: (i, 0)),
            pl.BlockSpec((IN_DIM, D), lambda i: (0, 0)),
            pl.BlockSpec((1, D), lambda i: (0, 0)),
        ],
        out_specs=pl.BlockSpec((BN, 2 * D), lambda i: (i, 0)),
        out_shape=jax.ShapeDtypeStruct((N, 2 * D), jnp.float32),
    )(features, W_enc, b_enc.reshape(1, D))


def _enc_if_body(sums_ref, x2_ref, z_ref, dinv_ref):
    deg = sums_ref[0][:, D:D + 1] + sums_ref[1][:, D:D + 1]
    dinv = 1.0 / jnp.maximum(deg, 1.0)
    agg = (sums_ref[0][:, :D] + sums_ref[1][:, :D]) * dinv
    v = jnp.zeros_like(agg)
    sp = []
    for _ in range(T):
        v = v + agg
        s = _spike(v)
        v = v - s * VTH
        sp.append(s)
    x2_ref[0] = jnp.concatenate([sp[0], sp[1]], axis=1)
    x2_ref[1] = jnp.concatenate([sp[2], sp[3]], axis=1)
    z_ref[...] = STEP * (sp[0] + sp[1] + sp[2] + sp[3])
    dinv_ref[...] = dinv


def _enc_if(sums):
    return pl.pallas_call(
        _enc_if_body,
        grid=(GRID,),
        in_specs=[
            pl.BlockSpec((2, BN, IN_DIM), lambda i: (0, i, 0)),
        ],
        out_specs=[
            pl.BlockSpec((2, BN, 2 * D), lambda i: (0, i, 0)),
            pl.BlockSpec((BN, D), lambda i: (i, 0)),
            pl.BlockSpec((BN, 1), lambda i: (i, 0)),
        ],
        out_shape=[
            jax.ShapeDtypeStruct((2, N, 2 * D), jnp.float32),
            jax.ShapeDtypeStruct((N, D), jnp.float32),
            jax.ShapeDtypeStruct((N, 1), jnp.float32),
        ],
    )(sums)


def _layer_body(m2_ref, dinv_ref, w_ref, b_ref, z_ref, x2_ref, zn_ref):
    dinv = dinv_ref[...]
    v = jnp.zeros((BN, D), jnp.float32)
    sp = []
    for t in range(T):
        m_t = m2_ref[t // 2][:, (t % 2) * D:(t % 2 + 1) * D] * dinv
        u = jnp.dot(m_t, w_ref[...], preferred_element_type=jnp.float32) + b_ref[...]
        v = v + u
        s = _spike(v)
        v = v - s * VTH
        sp.append(s)
    x2_ref[0] = jnp.concatenate([sp[0], sp[1]], axis=1)
    x2_ref[1] = jnp.concatenate([sp[2], sp[3]], axis=1)
    zn_ref[...] = z_ref[...] + STEP * (sp[0] + sp[1] + sp[2] + sp[3])


def _layer_tc(m2, dinv, W, b, z):
    return pl.pallas_call(
        _layer_body,
        grid=(GRID,),
        in_specs=[
            pl.BlockSpec((2, BN, 2 * D), lambda i: (0, i, 0)),
            pl.BlockSpec((BN, 1), lambda i: (i, 0)),
            pl.BlockSpec((D, D), lambda i: (0, 0)),
            pl.BlockSpec((1, D), lambda i: (0, 0)),
            pl.BlockSpec((BN, D), lambda i: (i, 0)),
        ],
        out_specs=[
            pl.BlockSpec((2, BN, 2 * D), lambda i: (0, i, 0)),
            pl.BlockSpec((BN, D), lambda i: (i, 0)),
        ],
        out_shape=[
            jax.ShapeDtypeStruct((2, N, 2 * D), jnp.float32),
            jax.ShapeDtypeStruct((N, D), jnp.float32),
        ],
    )(m2, dinv, W, b.reshape(1, D), z)


def _final_body(m2_ref, dinv_ref, w_ref, b_ref, z_ref, fcw_ref, fcb_ref, o_ref):
    dinv = dinv_ref[...]
    v = jnp.zeros((BN, D), jnp.float32)
    acc = jnp.zeros((BN, D), jnp.float32)
    for t in range(T):
        m_t = m2_ref[t // 2][:, (t % 2) * D:(t % 2 + 1) * D] * dinv
        u = jnp.dot(m_t, w_ref[...], preferred_element_type=jnp.float32) + b_ref[...]
        v = v + u
        s = _spike(v)
        v = v - s * VTH
        acc = acc + s
    zf = z_ref[...] + STEP * acc
    o_ref[...] = (
        jnp.dot(zf, fcw_ref[...], preferred_element_type=jnp.float32) + fcb_ref[...]
    )


def _final_tc(m2, dinv, W, b, z, fc_W, fc_b):
    return pl.pallas_call(
        _final_body,
        grid=(GRID,),
        in_specs=[
            pl.BlockSpec((2, BN, 2 * D), lambda i: (0, i, 0)),
            pl.BlockSpec((BN, 1), lambda i: (i, 0)),
            pl.BlockSpec((D, D), lambda i: (0, 0)),
            pl.BlockSpec((1, D), lambda i: (0, 0)),
            pl.BlockSpec((BN, D), lambda i: (i, 0)),
            pl.BlockSpec((D, C), lambda i: (0, 0)),
            pl.BlockSpec((1, C), lambda i: (0, 0)),
        ],
        out_specs=pl.BlockSpec((BN, C), lambda i: (i, 0)),
        out_shape=jax.ShapeDtypeStruct((N, C), jnp.float32),
    )(m2, dinv, W, b.reshape(1, D), z, fc_W, fc_b.reshape(1, C))


_MESH = plsc.VectorSubcoreMesh(core_axis_name="c", subcore_axis_name="s")


def _zero_stripe(shared, s, tbuf):
    def body(k, _):
        pltpu.sync_copy(tbuf, shared.at[pl.ds(s * RPT + k * RCH, RCH)])
        return 0

    lax.fori_loop(0, jnp.where(s == NTILES - 1, 5, 8), body, 0)


def _stripe_writeback(shared, hbm, s, tbuf):
    def body(k, _):
        off = s * RPT + k * RCH
        pltpu.sync_copy(shared.at[pl.ds(off, RCH)], tbuf)
        pltpu.sync_copy(tbuf, hbm.at[pl.ds(off, RCH)])
        return 0

    lax.fori_loop(0, jnp.where(s == NTILES - 1, 5, 8), body, 0)


def _block_pair(table, accum, src_hbm, dst_hbm, base, idx, rows, sems, isem):
    sblk0, dblk0, sblk1, dblk1 = idx
    pltpu.sync_copy(src_hbm.at[pl.ds(base, BLK)], sblk0)
    pltpu.sync_copy(dst_hbm.at[pl.ds(base, BLK)], dblk0)
    pf = [pltpu.async_copy(src_hbm.at[pl.ds(base + BLK, BLK)], sblk1, isem),
          pltpu.async_copy(dst_hbm.at[pl.ds(base + BLK, BLK)], dblk1, isem)]
    _pipelined_block(table, accum, sblk0, dblk0, rows, sems)
    pf[0].wait()
    pf[1].wait()
    _pipelined_block(table, accum, sblk1, dblk1, rows, sems)


def _pipelined_block(table, accum, sblk, dblk, rows, sems):
    gsem, _ = sems
    for half in range(BLK // 2):
        gd = [pltpu.async_copy(table.at[sblk.at[2 * half + k]], rows[k], gsem)
              for k in range(2)]
        for k in range(2):
            gd[k].wait()
            pltpu.sync_copy(rows[k], accum.at[dblk.at[2 * half + k]], add=True)


@functools.partial(
    pl.kernel,
    mesh=_MESH,
    out_type=jax.ShapeDtypeStruct((2, N, IN_DIM), jnp.float32),
    scratch_types=[
        pltpu.VMEM((BLK, CH), jnp.int32),
        pltpu.VMEM((BLK, CH), jnp.int32),
        pltpu.VMEM((BLK, CH), jnp.int32),
        pltpu.VMEM((BLK, CH), jnp.int32),
        pltpu.VMEM((CH, IN_DIM), jnp.float32),
        pltpu.VMEM((CH, IN_DIM), jnp.float32),
        pltpu.VMEM((RCH, IN_DIM), jnp.float32),
        pltpu.VMEM_SHARED((N + BLK, IN_DIM), jnp.float32),
        pltpu.SemaphoreType.DMA,
        pltpu.SemaphoreType.DMA,
        pltpu.SemaphoreType.DMA,
    ],
)
def _enc_agg_sc(f_hbm, src_hbm, dst_hbm, zrow_hbm, sums_hbm,
                sblk0, dblk0, sblk1, dblk1, rows0, rows1, tbuf, accum,
                sem0, sem1, isem):
    c = lax.axis_index("c")
    s = lax.axis_index("s")

    pltpu.sync_copy(zrow_hbm, tbuf)
    _zero_stripe(accum, s, tbuf)
    plsc.subcore_barrier()

    def body(pair, _):
        base = c * ECH_ENC + s * 80 + pair * 2 * BLK
        _block_pair(f_hbm, accum, src_hbm, dst_hbm, base,
                    (sblk0, dblk0, sblk1, dblk1), (rows0, rows1),
                    (sem0, sem1), isem)
        return 0

    lax.fori_loop(0, 5, body, 0)
    plsc.subcore_barrier()

    _stripe_writeback(accum, sums_hbm.at[c], s, tbuf)


@functools.partial(
    pl.kernel,
    mesh=_MESH,
    out_type=jax.ShapeDtypeStruct((2, N, 2 * D), jnp.float32),
    scratch_types=[
        pltpu.VMEM((BLK, CH), jnp.int32),
        pltpu.VMEM((BLK, CH), jnp.int32),
        pltpu.VMEM((BLK, CH), jnp.int32),
        pltpu.VMEM((BLK, CH), jnp.int32),
        pltpu.VMEM((CH, 2 * D), jnp.float32),
        pltpu.VMEM((CH, 2 * D), jnp.float32),
        pltpu.VMEM((RCH, 2 * D), jnp.float32),
        pltpu.VMEM_SHARED((N + BLK, 2 * D), jnp.float32),
        pltpu.SemaphoreType.DMA,
        pltpu.SemaphoreType.DMA,
        pltpu.SemaphoreType.DMA,
    ],
)
def _layer_agg_sc(x2_hbm, src_hbm, dst_hbm, zrow_hbm, m2_hbm,
                  sblk0, dblk0, sblk1, dblk1, rows0, rows1, tbuf, accum,
                  sem0, sem1, isem):
    c = lax.axis_index("c")
    s = lax.axis_index("s")

    pltpu.sync_copy(zrow_hbm, tbuf)
    _zero_stripe(accum, s, tbuf)
    plsc.subcore_barrier()

    def body(pair, _):
        base = s * 160 + pair * 2 * BLK
        _block_pair(x2_hbm.at[c], accum, src_hbm, dst_hbm, base,
                    (sblk0, dblk0, sblk1, dblk1), (rows0, rows1),
                    (sem0, sem1), isem)
        return 0

    lax.fori_loop(0, 10, body, 0)
    plsc.subcore_barrier()

    _stripe_writeback(accum, m2_hbm.at[c], s, tbuf)


@jax.jit
def kernel(features, edge_index, W_enc, b_enc, W_layers, b_layers, fc_W, fc_b):
    src = edge_index[0]
    dst = edge_index[1]
    zrow = jnp.zeros((RCH, IN_DIM), jnp.float32)

    fsrc_l = jnp.arange(FAKE_LAY, dtype=jnp.int32) % N
    fdst_l = N + jnp.arange(FAKE_LAY, dtype=jnp.int32) % BLK
    lsrc = jnp.concatenate([src, fsrc_l]).reshape(-1, CH)
    ldst = jnp.concatenate([dst, fdst_l]).reshape(-1, CH)

    h = _enc_matmul(features, W_enc, b_enc)
    sums = _enc_agg_sc(h, lsrc, ldst, zrow)
    x2, z, dinv = _enc_if(sums)

    m2 = _layer_agg_sc(x2, lsrc, ldst, zrow)
    x2, z = _layer_tc(m2, dinv, W_layers[0], b_layers[0], z)

    m2 = _layer_agg_sc(x2, lsrc, ldst, zrow)
    out = _final_tc(m2, dinv, W_layers[1], b_layers[1], z, fc_W, fc_b)
    return out
